# baseline (speedup 1.0000x reference)
import functools

import jax
import jax.numpy as jnp
from jax import lax
from jax.experimental import pallas as pl
from jax.experimental.pallas import tpu as pltpu
from jax.experimental.pallas import tpu_sc as plsc

NC = 2
NS = 16
NW = NC * NS

D = 32
C = 128
K = 4
CT = D // 8


def _gather(table, idx):
    nblk = idx.shape[0]
    per_w = nblk // NW
    ngrp = per_w // K
    assert ngrp % 2 == 0
    H = nblk // C
    mesh = plsc.VectorSubcoreMesh(core_axis_name="c", subcore_axis_name="s")

    @functools.partial(
        pl.kernel,
        mesh=mesh,
        out_type=jax.ShapeDtypeStruct((H, CT, C, 8, 128), jnp.float32),
        scratch_types=[
            pltpu.VMEM((2, K, C), jnp.int32),
            pltpu.VMEM((2, K, C, D), jnp.float32),
            pltpu.VMEM((2, K, CT, 8, 128), jnp.float32),
            [pltpu.SemaphoreType.DMA] * 2,
            [pltpu.SemaphoreType.DMA] * 2,
            [pltpu.SemaphoreType.DMA] * 2,
        ],
        compiler_params=pltpu.CompilerParams(
            use_tc_tiling_on_sc=False, needs_layout_passes=False
        ),
    )
    def k(table_hbm, idx_hbm, out_hbm, idx_v, rows_v, tiles_v, isems, gsems, osems):
        wid = lax.axis_index("s") * NC + lax.axis_index("c")
        wbase = wid * per_w
        riota = lax.iota(jnp.int32, 16)
        cvecs = [jnp.full((16,), c, jnp.int32) for c in range(D)]

        def step(g, p, q):
            gbase = wbase + g * K

            pltpu.make_async_copy(
                idx_hbm.at[pl.ds(gbase, K)], idx_v.at[p], isems[p]
            ).wait()

            for j in range(K):
                pltpu.async_copy(
                    table_hbm.at[idx_v.at[p, j]], rows_v.at[p, j], gsems[p]
                )

            @pl.when(g >= 3)
            def _():
                for j in range(K):
                    for ct in range(CT):
                        pltpu.make_async_copy(
                            tiles_v.at[q, j, ct], out_hbm.at[0, 0, 0], osems[q]
                        ).wait()

            @pl.when(g >= 1)
            def _():
                pbase = gbase - K
                for j in range(K):
                    pltpu.make_async_copy(
                        table_hbm.at[idx_v.at[q, j]], rows_v.at[q, j], gsems[q]
                    ).wait()

                    def tr(ch, carry):
                        rv = riota + 16 * ch
                        base = 16 * ch
                        for c in range(D):
                            vals = plsc.load_gather(
                                rows_v.at[q, j], [rv, cvecs[c]]
                            )
                            tiles_v[q, j, c // 8, c % 8, pl.ds(base, 16)] = vals
                        return carry

                    lax.fori_loop(0, 8, tr, 0)

                    bid = pbase + j
                    l = bid // C
                    bt = bid - l * C
                    for ct in range(CT):
                        pltpu.async_copy(
                            tiles_v.at[q, j, ct], out_hbm.at[l, ct, bt], osems[q]
                        )

            @pl.when(g + 1 < ngrp)
            def _():
                pltpu.async_copy(
                    idx_hbm.at[pl.ds(gbase + K, K)], idx_v.at[q], isems[q]
                )

        pltpu.async_copy(idx_hbm.at[pl.ds(wbase, K)], idx_v.at[0], isems[0])

        def grp2(g2, carry):
            step(2 * g2, 0, 1)
            step(2 * g2 + 1, 1, 0)
            return carry

        lax.fori_loop(0, ngrp // 2, grp2, 0)

        for j in range(K):
            for ct in range(CT):
                pltpu.make_async_copy(
                    tiles_v.at[1, j, ct], out_hbm.at[0, 0, 0], osems[1]
                ).wait()
        lbase = wbase + (ngrp - 1) * K
        for j in range(K):
            pltpu.make_async_copy(
                table_hbm.at[idx_v.at[1, j]], rows_v.at[1, j], gsems[1]
            ).wait()

            def tr(ch, carry):
                rv = riota + 16 * ch
                base = 16 * ch
                for c in range(D):
                    vals = plsc.load_gather(rows_v.at[1, j], [rv, cvecs[c]])
                    tiles_v[1, j, c // 8, c % 8, pl.ds(base, 16)] = vals
                return carry

            lax.fori_loop(0, 8, tr, 0)

            bid = lbase + j
            l = bid // C
            bt = bid - l * C
            for ct in range(CT):
                pltpu.async_copy(
                    tiles_v.at[1, j, ct], out_hbm.at[l, ct, bt], osems[1]
                )
        for par in (0, 1):
            for j in range(K):
                for ct in range(CT):
                    pltpu.make_async_copy(
                        tiles_v.at[par, j, ct], out_hbm.at[0, 0, 0], osems[par]
                    ).wait()

    return k(table, idx)


def kernel(indices, table):
    B, H = indices.shape
    idx2 = jnp.transpose(indices).reshape(H * (B // C), C).astype(jnp.int32)
    out5 = _gather(table, idx2)
    return out5.transpose(2, 4, 0, 1, 3).reshape(B, H, D)


# speedup vs baseline: 1.4210x; 1.4210x over previous
import functools

import jax
import jax.numpy as jnp
from jax import lax
from jax.experimental import pallas as pl
from jax.experimental.pallas import tpu as pltpu
from jax.experimental.pallas import tpu_sc as plsc

NC = 2
NS = 16
NW = NC * NS

D = 32
C = 128
K = 4
CT = D // 8


def _gather(table, idx):
    DP = table.shape[1]
    nblk = idx.shape[0]
    per_w = nblk // NW
    ngrp = per_w // K
    assert ngrp % 2 == 0
    H = nblk // C
    mesh = plsc.VectorSubcoreMesh(core_axis_name="c", subcore_axis_name="s")

    @functools.partial(
        pl.kernel,
        mesh=mesh,
        out_type=jax.ShapeDtypeStruct((H, CT, C, 8, 128), jnp.float32),
        scratch_types=[
            pltpu.VMEM((2, K, C), jnp.int32),
            pltpu.VMEM((2, K, C, DP), jnp.float32),
            pltpu.VMEM((2, K, CT, 8, 128), jnp.float32),
            [pltpu.SemaphoreType.DMA] * 2,
            [pltpu.SemaphoreType.DMA] * 2,
            [pltpu.SemaphoreType.DMA] * 2,
        ],
        compiler_params=pltpu.CompilerParams(
            use_tc_tiling_on_sc=False, needs_layout_passes=False
        ),
    )
    def k(table_hbm, idx_hbm, out_hbm, idx_v, rows_v, tiles_v, isems, gsems, osems):
        wid = lax.axis_index("s") * NC + lax.axis_index("c")
        wbase = wid * per_w
        riota = lax.iota(jnp.int32, 16)
        cvecs = [jnp.full((16,), c, jnp.int32) for c in range(D)]

        def step(g, p, q):
            gbase = wbase + g * K

            pltpu.make_async_copy(
                idx_hbm.at[pl.ds(gbase, K)], idx_v.at[p], isems[p]
            ).wait()

            for j in range(K):
                pltpu.async_copy(
                    table_hbm.at[idx_v.at[p, j]], rows_v.at[p, j], gsems[p]
                )

            @pl.when(g >= 3)
            def _():
                for j in range(K):
                    for ct in range(CT):
                        pltpu.make_async_copy(
                            tiles_v.at[q, j, ct], out_hbm.at[0, 0, 0], osems[q]
                        ).wait()

            @pl.when(g >= 1)
            def _():
                pbase = gbase - K
                for j in range(K):
                    pltpu.make_async_copy(
                        table_hbm.at[idx_v.at[q, j]], rows_v.at[q, j], gsems[q]
                    ).wait()

                    def tr(ch, carry):
                        rv = riota + 16 * ch
                        base = 16 * ch
                        for c in range(D):
                            vals = plsc.load_gather(
                                rows_v.at[q, j], [rv, cvecs[c]]
                            )
                            tiles_v[q, j, c // 8, c % 8, pl.ds(base, 16)] = vals
                        return carry

                    lax.fori_loop(0, 8, tr, 0)

                    bid = pbase + j
                    l = bid // C
                    bt = bid - l * C
                    for ct in range(CT):
                        pltpu.async_copy(
                            tiles_v.at[q, j, ct], out_hbm.at[l, ct, bt], osems[q]
                        )

            @pl.when(g + 1 < ngrp)
            def _():
                pltpu.async_copy(
                    idx_hbm.at[pl.ds(gbase + K, K)], idx_v.at[q], isems[q]
                )

        pltpu.async_copy(idx_hbm.at[pl.ds(wbase, K)], idx_v.at[0], isems[0])

        def grp2(g2, carry):
            step(2 * g2, 0, 1)
            step(2 * g2 + 1, 1, 0)
            return carry

        lax.fori_loop(0, ngrp // 2, grp2, 0)

        for j in range(K):
            for ct in range(CT):
                pltpu.make_async_copy(
                    tiles_v.at[1, j, ct], out_hbm.at[0, 0, 0], osems[1]
                ).wait()
        lbase = wbase + (ngrp - 1) * K
        for j in range(K):
            pltpu.make_async_copy(
                table_hbm.at[idx_v.at[1, j]], rows_v.at[1, j], gsems[1]
            ).wait()

            def tr(ch, carry):
                rv = riota + 16 * ch
                base = 16 * ch
                for c in range(D):
                    vals = plsc.load_gather(rows_v.at[1, j], [rv, cvecs[c]])
                    tiles_v[1, j, c // 8, c % 8, pl.ds(base, 16)] = vals
                return carry

            lax.fori_loop(0, 8, tr, 0)

            bid = lbase + j
            l = bid // C
            bt = bid - l * C
            for ct in range(CT):
                pltpu.async_copy(
                    tiles_v.at[1, j, ct], out_hbm.at[l, ct, bt], osems[1]
                )
        for par in (0, 1):
            for j in range(K):
                for ct in range(CT):
                    pltpu.make_async_copy(
                        tiles_v.at[par, j, ct], out_hbm.at[0, 0, 0], osems[par]
                    ).wait()

    return k(table, idx)


def kernel(indices, table):
    B, H = indices.shape
    idx2 = jnp.transpose(indices).reshape(H * (B // C), C).astype(jnp.int32)
    out5 = _gather(jnp.pad(table, ((0, 0), (0, 1))), idx2)
    return out5.transpose(2, 4, 0, 1, 3).reshape(B, H, D)


# speedup vs baseline: 2.1985x; 1.5471x over previous
import functools

import jax
import jax.numpy as jnp
from jax import lax
from jax.experimental import pallas as pl
from jax.experimental.pallas import tpu as pltpu
from jax.experimental.pallas import tpu_sc as plsc

NC = 2
NS = 16
NW = NC * NS

D = 32
C = 128
K = 4
CT = D // 8
TP = 129


def _gather(table, idx):
    DP = table.shape[1]
    nblk = idx.shape[0]
    per_w = nblk // NW
    ngrp = per_w // K
    assert ngrp % 2 == 0
    H = nblk // C
    mesh = plsc.VectorSubcoreMesh(core_axis_name="c", subcore_axis_name="s")

    @functools.partial(
        pl.kernel,
        mesh=mesh,
        out_type=jax.ShapeDtypeStruct((H, CT, C, 8, 128), jnp.float32),
        scratch_types=[
            pltpu.VMEM((2, K, C), jnp.int32),
            pltpu.VMEM((2, K, C, DP), jnp.float32),
            pltpu.VMEM((2, K, CT, 8, TP), jnp.float32),
            [pltpu.SemaphoreType.DMA] * 2,
            [pltpu.SemaphoreType.DMA] * 2,
            [pltpu.SemaphoreType.DMA] * 2,
        ],
        compiler_params=pltpu.CompilerParams(
            use_tc_tiling_on_sc=False, needs_layout_passes=False
        ),
    )
    def k(table_hbm, idx_hbm, out_hbm, idx_v, rows_v, tiles_v, isems, gsems, osems):
        wid = lax.axis_index("s") * NC + lax.axis_index("c")
        wbase = wid * per_w
        riota = lax.iota(jnp.int32, 16)
        ctvs, crvs = [], []
        for k2 in range(D // 16):
            ci = riota + 16 * k2
            ctv = ci // 8
            ctvs.append(ctv)
            crvs.append(ci - ctv * 8)

        def transpose_block(par, j):
            qv = jnp.full((16,), par, jnp.int32)
            jv = jnp.full((16,), j, jnp.int32)

            def tr(it, carry):
                for u in range(4):
                    br = 4 * it + u
                    brv = jnp.full((16,), br, jnp.int32)
                    for k2 in range(D // 16):
                        vals = rows_v[par, j, br, pl.ds(16 * k2, 16)]
                        plsc.store_scatter(
                            tiles_v, [qv, jv, ctvs[k2], crvs[k2], brv], vals
                        )
                return carry

            lax.fori_loop(0, C // 4, tr, 0)

        def step(g, p, q):
            gbase = wbase + g * K

            pltpu.make_async_copy(
                idx_hbm.at[pl.ds(gbase, K)], idx_v.at[p], isems[p]
            ).wait()

            for j in range(K):
                pltpu.async_copy(
                    table_hbm.at[idx_v.at[p, j]], rows_v.at[p, j], gsems[p]
                )

            @pl.when(g >= 3)
            def _():
                for j in range(K):
                    for ct in range(CT):
                        pltpu.make_async_copy(
                            tiles_v.at[q, j, ct, :, pl.ds(0, 128)],
                            out_hbm.at[0, 0, 0],
                            osems[q],
                        ).wait()

            @pl.when(g >= 1)
            def _():
                pbase = gbase - K
                for j in range(K):
                    pltpu.make_async_copy(
                        table_hbm.at[idx_v.at[q, j]], rows_v.at[q, j], gsems[q]
                    ).wait()

                    transpose_block(q, j)

                    bid = pbase + j
                    l = bid // C
                    bt = bid - l * C
                    for ct in range(CT):
                        pltpu.async_copy(
                            tiles_v.at[q, j, ct, :, pl.ds(0, 128)],
                            out_hbm.at[l, ct, bt],
                            osems[q],
                        )

            @pl.when(g + 1 < ngrp)
            def _():
                pltpu.async_copy(
                    idx_hbm.at[pl.ds(gbase + K, K)], idx_v.at[q], isems[q]
                )

        pltpu.async_copy(idx_hbm.at[pl.ds(wbase, K)], idx_v.at[0], isems[0])

        def grp2(g2, carry):
            step(2 * g2, 0, 1)
            step(2 * g2 + 1, 1, 0)
            return carry

        lax.fori_loop(0, ngrp // 2, grp2, 0)

        for j in range(K):
            for ct in range(CT):
                pltpu.make_async_copy(
                    tiles_v.at[1, j, ct, :, pl.ds(0, 128)],
                    out_hbm.at[0, 0, 0],
                    osems[1],
                ).wait()
        lbase = wbase + (ngrp - 1) * K
        for j in range(K):
            pltpu.make_async_copy(
                table_hbm.at[idx_v.at[1, j]], rows_v.at[1, j], gsems[1]
            ).wait()

            transpose_block(1, j)

            bid = lbase + j
            l = bid // C
            bt = bid - l * C
            for ct in range(CT):
                pltpu.async_copy(
                    tiles_v.at[1, j, ct, :, pl.ds(0, 128)],
                    out_hbm.at[l, ct, bt],
                    osems[1],
                )
        for par in (0, 1):
            for j in range(K):
                for ct in range(CT):
                    pltpu.make_async_copy(
                        tiles_v.at[par, j, ct, :, pl.ds(0, 128)],
                        out_hbm.at[0, 0, 0],
                        osems[par],
                    ).wait()

    return k(table, idx)


def kernel(indices, table):
    B, H = indices.shape
    idx2 = jnp.transpose(indices).reshape(H * (B // C), C).astype(jnp.int32)
    out5 = _gather(table, idx2)
    return out5.transpose(2, 4, 0, 1, 3).reshape(B, H, D)
